# initial kernel scaffold (unmeasured)
import jax
import jax.numpy as jnp
from jax import lax
from jax.experimental import pallas as pl
from jax.experimental.pallas import tpu as pltpu


def kernel(
    x,
):
    def body(*refs):
        pass

    out_shape = jax.ShapeDtypeStruct(..., jnp.float32)
    return pl.pallas_call(body, out_shape=out_shape)(...)


# baseline (device time: 330525 ns/iter reference)
import jax
import jax.numpy as jnp
from jax import lax
from jax.experimental import pallas as pl
from jax.experimental.pallas import tpu as pltpu

K = 4
M = 8192
N = 1024
MC = M // K


def kernel(x):
    x16 = x.astype(jnp.bfloat16)

    def body(x_ref, out_ref, sbuf, rbuf, gbuf, send_sems, recv_sems):
        my_x = lax.axis_index("x")
        my_y = lax.axis_index("y")
        my_z = lax.axis_index("z")
        nxt = (my_z + 1) % K
        prv = (my_z + K - 1) % K

        barrier_sem = pltpu.get_barrier_semaphore()
        for nbr in (prv, nxt):
            pl.semaphore_signal(
                barrier_sem, inc=1,
                device_id=(my_x, my_y, nbr),
                device_id_type=pl.DeviceIdType.MESH,
            )
        pl.semaphore_wait(barrier_sem, 2)

        sbuf[...] = x_ref[pl.ds(my_z * MC, MC), :]
        for s in range(K - 1):
            rdma = pltpu.make_async_remote_copy(
                src_ref=sbuf,
                dst_ref=rbuf.at[s],
                send_sem=send_sems.at[s],
                recv_sem=recv_sems.at[s],
                device_id=(my_x, my_y, nxt),
                device_id_type=pl.DeviceIdType.MESH,
            )
            rdma.start()
            rdma.wait()
            recv_idx = (my_z + K - 1 - s) % K
            sbuf[...] = x_ref[pl.ds(recv_idx * MC, MC), :] + rbuf[s]

        out_ref[pl.ds(((my_z + 1) % K) * MC, MC), :] = sbuf[...]

        for t in range(K - 1):
            rdma = pltpu.make_async_remote_copy(
                src_ref=sbuf if t == 0 else gbuf.at[t - 1],
                dst_ref=gbuf.at[t],
                send_sem=send_sems.at[K - 1 + t],
                recv_sem=recv_sems.at[K - 1 + t],
                device_id=(my_x, my_y, nxt),
                device_id_type=pl.DeviceIdType.MESH,
            )
            rdma.start()
            rdma.wait()
            out_ref[pl.ds(((my_z + K - t) % K) * MC, MC), :] = gbuf[t]

    return pl.pallas_call(
        body,
        out_shape=jax.ShapeDtypeStruct((M, N), jnp.bfloat16),
        in_specs=[pl.BlockSpec(memory_space=pltpu.VMEM)],
        out_specs=pl.BlockSpec(memory_space=pltpu.VMEM),
        scratch_shapes=[
            pltpu.VMEM((MC, N), jnp.bfloat16),
            pltpu.VMEM((K - 1, MC, N), jnp.bfloat16),
            pltpu.VMEM((K - 1, MC, N), jnp.bfloat16),
            pltpu.SemaphoreType.DMA((2 * (K - 1),)),
            pltpu.SemaphoreType.DMA((2 * (K - 1),)),
        ],
        compiler_params=pltpu.CompilerParams(
            collective_id=0,
            vmem_limit_bytes=64 * 1024 * 1024,
        ),
    )(x16)


# device time: 185885 ns/iter; 1.7781x vs baseline; 1.7781x over previous
import jax
import jax.numpy as jnp
from jax import lax
from jax.experimental import pallas as pl
from jax.experimental.pallas import tpu as pltpu

K = 4
M = 8192
N = 1024
P = 8
MS = M // P
MCZ = MS // K
H = MS // 2


def kernel(x):
    x16 = x.astype(jnp.bfloat16)

    def body(x_ref, out_ref, sbuf, rbuf, gbuf, slice_ref, cw_ref, ccw_ref,
             zs_send, zs_recv, cw_send, cw_recv, ccw_send, ccw_recv):
        my_x = lax.axis_index("x")
        my_y = lax.axis_index("y")
        my_z = lax.axis_index("z")
        nxt_z = (my_z + 1) % K
        prv_z = (my_z + K - 1) % K
        p = my_x * 4 + my_y
        rr = jnp.where(my_x == 0, my_y, 7 - my_y)

        def ring_xy(r):
            rx = (r >= 4).astype(jnp.int32)
            ry = jnp.where(r < 4, r, 7 - r)
            return rx, ry

        nxt_rx, nxt_ry = ring_xy((rr + 1) % P)
        prv_rx, prv_ry = ring_xy((rr + P - 1) % P)

        barrier_sem = pltpu.get_barrier_semaphore()
        for dev in ((my_x, my_y, prv_z), (my_x, my_y, nxt_z),
                    (nxt_rx, nxt_ry, my_z), (prv_rx, prv_ry, my_z)):
            pl.semaphore_signal(
                barrier_sem, inc=1,
                device_id=dev, device_id_type=pl.DeviceIdType.MESH,
            )
        pl.semaphore_wait(barrier_sem, 4)

        base = p * MS

        sbuf[...] = x_ref[pl.ds(base + my_z * MCZ, MCZ), :]
        for s in range(K - 1):
            rdma = pltpu.make_async_remote_copy(
                src_ref=sbuf,
                dst_ref=rbuf.at[s],
                send_sem=zs_send.at[s],
                recv_sem=zs_recv.at[s],
                device_id=(my_x, my_y, nxt_z),
                device_id_type=pl.DeviceIdType.MESH,
            )
            rdma.start()
            rdma.wait()
            ridx = (my_z + K - 1 - s) % K
            sbuf[...] = x_ref[pl.ds(base + ridx * MCZ, MCZ), :] + rbuf[s]
        slice_ref[pl.ds(((my_z + 1) % K) * MCZ, MCZ), :] = sbuf[...]
        for t in range(K - 1):
            rdma = pltpu.make_async_remote_copy(
                src_ref=sbuf if t == 0 else gbuf.at[t - 1],
                dst_ref=gbuf.at[t],
                send_sem=zs_send.at[K - 1 + t],
                recv_sem=zs_recv.at[K - 1 + t],
                device_id=(my_x, my_y, nxt_z),
                device_id_type=pl.DeviceIdType.MESH,
            )
            rdma.start()
            rdma.wait()
            slice_ref[pl.ds(((my_z + K - t) % K) * MCZ, MCZ), :] = gbuf[t]
        out_ref[pl.ds(base, MS), :] = slice_ref[...]

        for h in range(P - 1):
            cw = pltpu.make_async_remote_copy(
                src_ref=slice_ref.at[pl.ds(0, H), :] if h == 0 else cw_ref.at[h - 1],
                dst_ref=cw_ref.at[h],
                send_sem=cw_send.at[h],
                recv_sem=cw_recv.at[h],
                device_id=(nxt_rx, nxt_ry, my_z),
                device_id_type=pl.DeviceIdType.MESH,
            )
            ccw = pltpu.make_async_remote_copy(
                src_ref=slice_ref.at[pl.ds(H, H), :] if h == 0 else ccw_ref.at[h - 1],
                dst_ref=ccw_ref.at[h],
                send_sem=ccw_send.at[h],
                recv_sem=ccw_recv.at[h],
                device_id=(prv_rx, prv_ry, my_z),
                device_id_type=pl.DeviceIdType.MESH,
            )
            cw.start()
            ccw.start()
            cw.wait()
            ccw.wait()
            o_cw = (rr + P - 1 - h) % P
            o_ccw = (rr + 1 + h) % P
            ox, oy = ring_xy(o_cw)
            out_ref[pl.ds((ox * 4 + oy) * MS, H), :] = cw_ref[h]
            ox2, oy2 = ring_xy(o_ccw)
            out_ref[pl.ds((ox2 * 4 + oy2) * MS + H, H), :] = ccw_ref[h]

    return pl.pallas_call(
        body,
        out_shape=jax.ShapeDtypeStruct((M, N), jnp.bfloat16),
        in_specs=[pl.BlockSpec(memory_space=pltpu.VMEM)],
        out_specs=pl.BlockSpec(memory_space=pltpu.VMEM),
        scratch_shapes=[
            pltpu.VMEM((MCZ, N), jnp.bfloat16),
            pltpu.VMEM((K - 1, MCZ, N), jnp.bfloat16),
            pltpu.VMEM((K - 1, MCZ, N), jnp.bfloat16),
            pltpu.VMEM((MS, N), jnp.bfloat16),
            pltpu.VMEM((P - 1, H, N), jnp.bfloat16),
            pltpu.VMEM((P - 1, H, N), jnp.bfloat16),
            pltpu.SemaphoreType.DMA((2 * (K - 1),)),
            pltpu.SemaphoreType.DMA((2 * (K - 1),)),
            pltpu.SemaphoreType.DMA((P - 1,)),
            pltpu.SemaphoreType.DMA((P - 1,)),
            pltpu.SemaphoreType.DMA((P - 1,)),
            pltpu.SemaphoreType.DMA((P - 1,)),
        ],
        compiler_params=pltpu.CompilerParams(
            collective_id=0,
            vmem_limit_bytes=64 * 1024 * 1024,
        ),
    )(x16)


# device time: 162322 ns/iter; 2.0362x vs baseline; 1.1452x over previous
import jax
import jax.numpy as jnp
from jax import lax
from jax.experimental import pallas as pl
from jax.experimental.pallas import tpu as pltpu

K = 4
M = 8192
N = 1024
P = 8
MS = M // P
MCZ = MS // K
H = MS // 2

BF = jnp.bfloat16


def kernel(x):
    def body(x_hbm, out_ref, xsl, sbuf, rbuf, gbuf, slice_ref, cw_ref,
             ccw_ref, load_sem, zs_send, zs_recv, ag_send, ag_recv,
             cw_send, cw_recv, ccw_send, ccw_recv):
        my_x = lax.axis_index("x")
        my_y = lax.axis_index("y")
        my_z = lax.axis_index("z")
        nxt_z = (my_z + 1) % K
        prv_z = (my_z + K - 1) % K
        p = my_x * 4 + my_y
        rr = jnp.where(my_x == 0, my_y, 7 - my_y)
        base = p * MS

        def ring_xy(r):
            rx = (r >= 4).astype(jnp.int32)
            ry = jnp.where(r < 4, r, 7 - r)
            return rx, ry

        nxt_rx, nxt_ry = ring_xy((rr + 1) % P)
        prv_rx, prv_ry = ring_xy((rr + P - 1) % P)

        load = pltpu.make_async_copy(
            x_hbm.at[pl.ds(base, MS), :], xsl, load_sem
        )
        load.start()

        barrier_sem = pltpu.get_barrier_semaphore()
        for dev in ((my_x, my_y, prv_z), (my_x, my_y, nxt_z),
                    (nxt_rx, nxt_ry, my_z), (prv_rx, prv_ry, my_z)):
            pl.semaphore_signal(
                barrier_sem, inc=1,
                device_id=dev, device_id_type=pl.DeviceIdType.MESH,
            )
        pl.semaphore_wait(barrier_sem, 4)
        load.wait()

        def xchunk(i):
            return xsl[pl.ds(i * MCZ, MCZ), :].astype(BF)

        def mk_rs(s):
            return pltpu.make_async_remote_copy(
                src_ref=sbuf.at[s % 2],
                dst_ref=rbuf.at[s],
                send_sem=zs_send.at[s],
                recv_sem=zs_recv.at[s],
                device_id=(my_x, my_y, nxt_z),
                device_id_type=pl.DeviceIdType.MESH,
            )

        sbuf[0] = xchunk(my_z)
        mk_rs(0).start()
        for s in range(K - 1):
            mk_rs(s).wait_recv()
            ridx = (my_z + K - 1 - s) % K
            nslot = (s + 1) % 2
            if s < K - 2:
                if s >= 1:
                    mk_rs(s - 1).wait_send()
                sbuf[nslot] = xchunk(ridx) + rbuf[s]
                mk_rs(s + 1).start()
            else:
                mk_rs(s - 1).wait_send()
                sbuf[1] = xchunk(ridx) + rbuf[s]

        def mk_ag(t):
            return pltpu.make_async_remote_copy(
                src_ref=sbuf.at[1] if t == 0 else gbuf.at[t - 1],
                dst_ref=gbuf.at[t],
                send_sem=ag_send.at[t],
                recv_sem=ag_recv.at[t],
                device_id=(my_x, my_y, nxt_z),
                device_id_type=pl.DeviceIdType.MESH,
            )

        mk_ag(0).start()
        slice_ref[pl.ds(((my_z + 1) % K) * MCZ, MCZ), :] = sbuf[1]
        for t in range(K - 1):
            mk_ag(t).wait_recv()
            if t < K - 2:
                mk_ag(t + 1).start()
            slice_ref[pl.ds(((my_z + K - t) % K) * MCZ, MCZ), :] = gbuf[t]

        def mk_cw(h):
            return pltpu.make_async_remote_copy(
                src_ref=slice_ref.at[pl.ds(0, H), :] if h == 0 else cw_ref.at[h - 1],
                dst_ref=cw_ref.at[h],
                send_sem=cw_send.at[h],
                recv_sem=cw_recv.at[h],
                device_id=(nxt_rx, nxt_ry, my_z),
                device_id_type=pl.DeviceIdType.MESH,
            )

        def mk_ccw(h):
            return pltpu.make_async_remote_copy(
                src_ref=slice_ref.at[pl.ds(H, H), :] if h == 0 else ccw_ref.at[h - 1],
                dst_ref=ccw_ref.at[h],
                send_sem=ccw_send.at[h],
                recv_sem=ccw_recv.at[h],
                device_id=(prv_rx, prv_ry, my_z),
                device_id_type=pl.DeviceIdType.MESH,
            )

        def store_xy(h):
            o_cw = (rr + P - 1 - h) % P
            o_ccw = (rr + 1 + h) % P
            ox, oy = ring_xy(o_cw)
            out_ref[pl.ds((ox * 4 + oy) * MS, H), :] = cw_ref[h]
            ox2, oy2 = ring_xy(o_ccw)
            out_ref[pl.ds((ox2 * 4 + oy2) * MS + H, H), :] = ccw_ref[h]

        mk_cw(0).start()
        mk_ccw(0).start()
        out_ref[pl.ds(base, MS), :] = slice_ref[...]
        for h in range(P - 1):
            mk_cw(h).wait_recv()
            mk_ccw(h).wait_recv()
            if h < P - 2:
                mk_cw(h + 1).start()
                mk_ccw(h + 1).start()
            store_xy(h)

        mk_rs(K - 2).wait_send()
        for t in range(K - 1):
            mk_ag(t).wait_send()
        for h in range(P - 1):
            mk_cw(h).wait_send()
            mk_ccw(h).wait_send()

    return pl.pallas_call(
        body,
        out_shape=jax.ShapeDtypeStruct((M, N), BF),
        in_specs=[pl.BlockSpec(memory_space=pl.ANY)],
        out_specs=pl.BlockSpec(memory_space=pltpu.VMEM),
        scratch_shapes=[
            pltpu.VMEM((MS, N), jnp.float32),
            pltpu.VMEM((2, MCZ, N), BF),
            pltpu.VMEM((K - 1, MCZ, N), BF),
            pltpu.VMEM((K - 1, MCZ, N), BF),
            pltpu.VMEM((MS, N), BF),
            pltpu.VMEM((P - 1, H, N), BF),
            pltpu.VMEM((P - 1, H, N), BF),
            pltpu.SemaphoreType.DMA,
            pltpu.SemaphoreType.DMA((K - 1,)),
            pltpu.SemaphoreType.DMA((K - 1,)),
            pltpu.SemaphoreType.DMA((K - 1,)),
            pltpu.SemaphoreType.DMA((K - 1,)),
            pltpu.SemaphoreType.DMA((P - 1,)),
            pltpu.SemaphoreType.DMA((P - 1,)),
            pltpu.SemaphoreType.DMA((P - 1,)),
            pltpu.SemaphoreType.DMA((P - 1,)),
        ],
        compiler_params=pltpu.CompilerParams(
            collective_id=0,
            vmem_limit_bytes=64 * 1024 * 1024,
        ),
    )(x)


# device time: 147247 ns/iter; 2.2447x vs baseline; 1.1024x over previous
import jax
import jax.numpy as jnp
from jax import lax
from jax.experimental import pallas as pl
from jax.experimental.pallas import tpu as pltpu

K = 4
M = 8192
N = 1024
P = 8
MS = M // P
MCZ = MS // K

BF = jnp.bfloat16


def kernel(x):
    def body(x_hbm, out_ref, xsl, sbuf, rbuf, gbuf, inj, st_ref, load_sem,
             zs_send, zs_recv, ag_send, ag_recv, st_send, st_recv):
        my_x = lax.axis_index("x")
        my_y = lax.axis_index("y")
        my_z = lax.axis_index("z")
        nxt_z = (my_z + 1) % K
        prv_z = (my_z + K - 1) % K
        p = my_x * 4 + my_y
        rr = jnp.where(my_x == 0, my_y, 7 - my_y)
        base = p * MS

        def ring_xy(r):
            rx = (r >= 4).astype(jnp.int32)
            ry = jnp.where(r < 4, r, 7 - r)
            return rx, ry

        nxt_rx, nxt_ry = ring_xy((rr + 1) % P)
        prv_rx, prv_ry = ring_xy((rr + P - 1) % P)

        load = pltpu.make_async_copy(
            x_hbm.at[pl.ds(base, MS), :], xsl, load_sem
        )
        load.start()

        barrier_sem = pltpu.get_barrier_semaphore()
        for dev in ((my_x, my_y, prv_z), (my_x, my_y, nxt_z),
                    (nxt_rx, nxt_ry, my_z), (prv_rx, prv_ry, my_z)):
            pl.semaphore_signal(
                barrier_sem, inc=1,
                device_id=dev, device_id_type=pl.DeviceIdType.MESH,
            )
        pl.semaphore_wait(barrier_sem, 4)
        load.wait()

        def xchunk(i):
            return xsl[pl.ds(i * MCZ, MCZ), :].astype(BF)

        def mk_st(c, h):
            cw = c < K // 2
            return pltpu.make_async_remote_copy(
                src_ref=inj.at[c] if h == 0 else st_ref.at[c, h - 1],
                dst_ref=st_ref.at[c, h],
                send_sem=st_send.at[c, h],
                recv_sem=st_recv.at[c, h],
                device_id=(nxt_rx, nxt_ry, my_z) if cw else (prv_rx, prv_ry, my_z),
                device_id_type=pl.DeviceIdType.MESH,
            )

        def inject(cid, val_ref):
            out_ref[pl.ds(base + cid * MCZ, MCZ), :] = val_ref[...]
            for c in range(K):
                @pl.when(cid == c)
                def _():
                    inj[c] = val_ref[...]
                    mk_st(c, 0).start()

        def mk_rs(s):
            return pltpu.make_async_remote_copy(
                src_ref=sbuf.at[s % 2],
                dst_ref=rbuf.at[s],
                send_sem=zs_send.at[s],
                recv_sem=zs_recv.at[s],
                device_id=(my_x, my_y, nxt_z),
                device_id_type=pl.DeviceIdType.MESH,
            )

        sbuf[0] = xchunk(my_z)
        mk_rs(0).start()
        for s in range(K - 1):
            mk_rs(s).wait_recv()
            ridx = (my_z + K - 1 - s) % K
            nslot = (s + 1) % 2
            if s < K - 2:
                if s >= 1:
                    mk_rs(s - 1).wait_send()
                sbuf[nslot] = xchunk(ridx) + rbuf[s]
                mk_rs(s + 1).start()
            else:
                mk_rs(s - 1).wait_send()
                sbuf[1] = xchunk(ridx) + rbuf[s]

        def mk_ag(t):
            return pltpu.make_async_remote_copy(
                src_ref=sbuf.at[1] if t == 0 else gbuf.at[t - 1],
                dst_ref=gbuf.at[t],
                send_sem=ag_send.at[t],
                recv_sem=ag_recv.at[t],
                device_id=(my_x, my_y, nxt_z),
                device_id_type=pl.DeviceIdType.MESH,
            )

        mk_ag(0).start()
        inject((my_z + 1) % K, sbuf.at[1])
        for t in range(K - 1):
            mk_ag(t).wait_recv()
            if t < K - 2:
                mk_ag(t + 1).start()
            inject((my_z + K - t) % K, gbuf.at[t])

        def store_st(c, h):
            o = (rr + P - 1 - h) % P if c < K // 2 else (rr + 1 + h) % P
            ox, oy = ring_xy(o)
            out_ref[pl.ds((ox * 4 + oy) * MS + c * MCZ, MCZ), :] = st_ref[c, h]

        for h in range(1, P - 1):
            for c in range(K):
                mk_st(c, h - 1).wait_recv()
                mk_st(c, h).start()
            for c in range(K):
                store_st(c, h - 1)
        for c in range(K):
            mk_st(c, P - 2).wait_recv()
            store_st(c, P - 2)

        mk_rs(K - 2).wait_send()
        for t in range(K - 1):
            mk_ag(t).wait_send()
        for c in range(K):
            for h in range(P - 1):
                mk_st(c, h).wait_send()

    return pl.pallas_call(
        body,
        out_shape=jax.ShapeDtypeStruct((M, N), BF),
        in_specs=[pl.BlockSpec(memory_space=pl.ANY)],
        out_specs=pl.BlockSpec(memory_space=pltpu.VMEM),
        scratch_shapes=[
            pltpu.VMEM((MS, N), jnp.float32),
            pltpu.VMEM((2, MCZ, N), BF),
            pltpu.VMEM((K - 1, MCZ, N), BF),
            pltpu.VMEM((K - 1, MCZ, N), BF),
            pltpu.VMEM((K, MCZ, N), BF),
            pltpu.VMEM((K, P - 1, MCZ, N), BF),
            pltpu.SemaphoreType.DMA,
            pltpu.SemaphoreType.DMA((K - 1,)),
            pltpu.SemaphoreType.DMA((K - 1,)),
            pltpu.SemaphoreType.DMA((K - 1,)),
            pltpu.SemaphoreType.DMA((K - 1,)),
            pltpu.SemaphoreType.DMA((K, P - 1)),
            pltpu.SemaphoreType.DMA((K, P - 1)),
        ],
        compiler_params=pltpu.CompilerParams(
            collective_id=0,
            vmem_limit_bytes=64 * 1024 * 1024,
        ),
    )(x)
